# baseline (speedup 1.0000x reference)
import functools

import jax
import jax.numpy as jnp
from jax import lax
from jax.experimental import pallas as pl
from jax.experimental.pallas import tpu as pltpu
from jax.experimental.pallas import tpu_sc as plsc

N_SPINS = 20
BATCH = 16384
NUM_WORKERS = 32
B_PER_W = BATCH // NUM_WORKERS
ROWS = 4
GROUPS_PER_ROW = 8


def _sc_body(sig_hbm, amps_hbm, out_hbm, sig_v, idx_v, out_v, sem):
    nc = 2
    wid = lax.axis_index("s") * nc + lax.axis_index("c")
    base = wid * B_PER_W

    pltpu.sync_copy(sig_hbm.at[pl.ds(base * N_SPINS, B_PER_W * N_SPINS)],
                    sig_v)

    lane = lax.iota(jnp.int32, 16) * N_SPINS
    zeros = jnp.zeros((16,), jnp.int32)

    copies = []
    for r in range(ROWS):
        def row_body(j, _, r=r):
            elem0 = (r * GROUPS_PER_ROW + j) * 16
            off = lane + elem0 * N_SPINS
            acc = zeros
            for i in range(N_SPINS):
                v = plsc.load_gather(sig_v, [off + i])
                acc = acc | jnp.where(v > 0.0,
                                      jnp.full((16,), 1 << i, jnp.int32),
                                      zeros)
            idx_v[r, pl.ds(j * 16, 16)] = acc
            return 0

        lax.fori_loop(0, GROUPS_PER_ROW, row_body, 0)
        copies.append(
            pltpu.async_copy(amps_hbm.at[idx_v.at[r]],
                             out_v.at[pl.ds(r * 128, 128)], sem))
    for cp in copies:
        cp.wait()

    pltpu.sync_copy(out_v, out_hbm.at[pl.ds(base, B_PER_W)])


@jax.jit
def kernel(sigma, amps):
    sig_flat = sigma.reshape(-1)
    mesh = plsc.VectorSubcoreMesh(core_axis_name="c", subcore_axis_name="s")
    k = functools.partial(
        pl.kernel,
        mesh=mesh,
        out_type=jax.ShapeDtypeStruct((BATCH,), jnp.float32),
        scratch_types=[
            pltpu.VMEM((B_PER_W * N_SPINS,), jnp.float32),
            pltpu.VMEM((ROWS, 128), jnp.int32),
            pltpu.VMEM((B_PER_W,), jnp.float32),
            pltpu.SemaphoreType.DMA,
        ],
        compiler_params=pltpu.CompilerParams(needs_layout_passes=False),
    )(_sc_body)
    return k(sig_flat, amps)


# speedup vs baseline: 1.0047x; 1.0047x over previous
import functools

import jax
import jax.numpy as jnp
from jax import lax
from jax.experimental import pallas as pl
from jax.experimental.pallas import tpu as pltpu
from jax.experimental.pallas import tpu_sc as plsc

N_SPINS = 20
BATCH = 16384
NUM_WORKERS = 32
B_PER_W = BATCH // NUM_WORKERS
ROWS = 4
GROUPS_PER_ROW = 8


def _sc_body(sig_hbm, amps_hbm, out_hbm, sig_v, idx_v, out_v, sem):
    nc = 2
    wid = lax.axis_index("s") * nc + lax.axis_index("c")
    base = wid * B_PER_W

    pltpu.sync_copy(sig_hbm.at[pl.ds(base * N_SPINS, B_PER_W * N_SPINS)],
                    sig_v)

    lane = lax.iota(jnp.int32, 16) * N_SPINS
    zeros = jnp.zeros((16,), jnp.int32)

    def group_body(g, _):
        off = lane + g * (16 * N_SPINS)
        acc = zeros
        for i in range(N_SPINS):
            v = plsc.load_gather(sig_v, [off + i])
            acc = acc | jnp.where(v > 0.0,
                                  jnp.full((16,), 1 << i, jnp.int32),
                                  zeros)
        idx_v[pl.ds(g * 16, 16)] = acc
        return 0

    lax.fori_loop(0, B_PER_W // 16, group_body, 0, unroll=False)

    copies = [
        pltpu.async_copy(amps_hbm.at[idx_v.at[pl.ds(r * 128, 128)]],
                         out_v.at[pl.ds(r * 128, 128)], sem)
        for r in range(ROWS)
    ]
    for cp in copies:
        cp.wait()

    pltpu.sync_copy(out_v, out_hbm.at[pl.ds(base, B_PER_W)])


@jax.jit
def kernel(sigma, amps):
    sig_flat = sigma.reshape(-1)
    mesh = plsc.VectorSubcoreMesh(core_axis_name="c", subcore_axis_name="s")
    k = functools.partial(
        pl.kernel,
        mesh=mesh,
        out_type=jax.ShapeDtypeStruct((BATCH,), jnp.float32),
        scratch_types=[
            pltpu.VMEM((B_PER_W * N_SPINS,), jnp.float32),
            pltpu.VMEM((B_PER_W,), jnp.int32),
            pltpu.VMEM((B_PER_W,), jnp.float32),
            pltpu.SemaphoreType.DMA,
        ],
        compiler_params=pltpu.CompilerParams(needs_layout_passes=False),
    )(_sc_body)
    return k(sig_flat, amps)


# speedup vs baseline: 1.1304x; 1.1251x over previous
import functools

import jax
import jax.numpy as jnp
from jax import lax
from jax.experimental import pallas as pl
from jax.experimental.pallas import tpu as pltpu
from jax.experimental.pallas import tpu_sc as plsc

N_SPINS = 20
BATCH = 16384
NUM_WORKERS = 32
B_PER_W = BATCH // NUM_WORKERS
ROWS = 4


def _sc_body(sig_hbm, amps_hbm, out_hbm, sig_v, idx_v, out_v, sem):
    nc = 2
    wid = lax.axis_index("s") * nc + lax.axis_index("c")
    base = wid * B_PER_W

    pltpu.sync_copy(sig_hbm.at[pl.ds(base, B_PER_W), :], sig_v)

    lane = lax.iota(jnp.int32, 16)
    zeros = jnp.zeros((16,), jnp.int32)

    def group_body(g, _):
        rows = lane + g * 16
        acc = zeros
        for i in range(N_SPINS):
            v = plsc.load_gather(sig_v, [rows, jnp.full((16,), i, jnp.int32)])
            acc = acc | jnp.where(v > 0.0,
                                  jnp.full((16,), 1 << i, jnp.int32),
                                  zeros)
        idx_v[pl.ds(g * 16, 16)] = acc
        return 0

    lax.fori_loop(0, B_PER_W // 16, group_body, 0, unroll=False)

    copies = [
        pltpu.async_copy(amps_hbm.at[idx_v.at[pl.ds(r * 128, 128)]],
                         out_v.at[pl.ds(r * 128, 128)], sem)
        for r in range(ROWS)
    ]
    for cp in copies:
        cp.wait()

    pltpu.sync_copy(out_v, out_hbm.at[pl.ds(base, B_PER_W)])


@jax.jit
def kernel(sigma, amps):
    mesh = plsc.VectorSubcoreMesh(core_axis_name="c", subcore_axis_name="s")
    k = functools.partial(
        pl.kernel,
        mesh=mesh,
        out_type=jax.ShapeDtypeStruct((BATCH,), jnp.float32),
        scratch_types=[
            pltpu.VMEM((B_PER_W, N_SPINS), jnp.float32),
            pltpu.VMEM((B_PER_W,), jnp.int32),
            pltpu.VMEM((B_PER_W,), jnp.float32),
            pltpu.SemaphoreType.DMA,
        ],
        compiler_params=pltpu.CompilerParams(needs_layout_passes=False),
    )(_sc_body)
    return k(sigma, amps)


# speedup vs baseline: 1.6167x; 1.4302x over previous
import functools

import jax
import jax.numpy as jnp
from jax import lax
from jax.experimental import pallas as pl
from jax.experimental.pallas import tpu as pltpu
from jax.experimental.pallas import tpu_sc as plsc

N_SPINS = 20
BATCH = 16384
NUM_WORKERS = 32
B_PER_W = BATCH // NUM_WORKERS
ROWS = 4


def _sc_body(sig_hbm, amps_hbm, out_hbm, sig_v, idx_v, out_v, sem):
    nc = 2
    wid = lax.axis_index("s") * nc + lax.axis_index("c")
    base = wid * B_PER_W

    pltpu.sync_copy(sig_hbm.at[:, pl.ds(base, B_PER_W)], sig_v)

    zeros = jnp.zeros((16,), jnp.int32)

    def group_body(g, _):
        c0 = g * 16
        acc = zeros
        for i in range(N_SPINS):
            v = sig_v[i, pl.ds(c0, 16)]
            acc = acc | jnp.where(v > 0.0,
                                  jnp.full((16,), 1 << i, jnp.int32),
                                  zeros)
        idx_v[pl.ds(c0, 16)] = acc
        return 0

    lax.fori_loop(0, B_PER_W // 16, group_body, 0, unroll=False)

    copies = [
        pltpu.async_copy(amps_hbm.at[idx_v.at[pl.ds(r * 128, 128)]],
                         out_v.at[pl.ds(r * 128, 128)], sem)
        for r in range(ROWS)
    ]
    for cp in copies:
        cp.wait()

    pltpu.sync_copy(out_v, out_hbm.at[pl.ds(base, B_PER_W)])


@jax.jit
def kernel(sigma, amps):
    sig_t = sigma.T
    mesh = plsc.VectorSubcoreMesh(core_axis_name="c", subcore_axis_name="s")
    k = functools.partial(
        pl.kernel,
        mesh=mesh,
        out_type=jax.ShapeDtypeStruct((BATCH,), jnp.float32),
        scratch_types=[
            pltpu.VMEM((N_SPINS, B_PER_W), jnp.float32),
            pltpu.VMEM((B_PER_W,), jnp.int32),
            pltpu.VMEM((B_PER_W,), jnp.float32),
            pltpu.SemaphoreType.DMA,
        ],
        compiler_params=pltpu.CompilerParams(needs_layout_passes=False),
    )(_sc_body)
    return k(sig_t, amps)
